# baseline (speedup 1.0000x reference)
import jax
import jax.numpy as jnp
from jax.experimental import pallas as pl
from jax.experimental.pallas import tpu as pltpu

_GRID = 8


def _copy3(idx_ref, r_ref, t_ref, p_ref, ro_ref, to_ref, po_ref):
    del idx_ref
    ro_ref[...] = r_ref[0]
    to_ref[...] = t_ref[0]
    po_ref[...] = p_ref[0]


def kernel(rule_prob, token_prob, reference_prob, length):
    L, B, R = rule_prob.shape
    V = token_prob.shape[2]
    M = reference_prob.shape[2]
    idx = (length - 1).astype(jnp.int32)
    Bb = B // _GRID

    grid_spec = pltpu.PrefetchScalarGridSpec(
        num_scalar_prefetch=1,
        grid=(_GRID,),
        in_specs=[
            pl.BlockSpec((1, Bb, R), lambda g, idx_ref: (idx_ref[0], g, 0)),
            pl.BlockSpec((1, Bb, V), lambda g, idx_ref: (idx_ref[0], g, 0)),
            pl.BlockSpec((1, Bb, M), lambda g, idx_ref: (idx_ref[0], g, 0)),
        ],
        out_specs=[
            pl.BlockSpec((Bb, R), lambda g, idx_ref: (g, 0)),
            pl.BlockSpec((Bb, V), lambda g, idx_ref: (g, 0)),
            pl.BlockSpec((Bb, M), lambda g, idx_ref: (g, 0)),
        ],
    )
    out = pl.pallas_call(
        _copy3,
        grid_spec=grid_spec,
        out_shape=[
            jax.ShapeDtypeStruct((B, R), jnp.float32),
            jax.ShapeDtypeStruct((B, V), jnp.float32),
            jax.ShapeDtypeStruct((B, M), jnp.float32),
        ],
    )(idx, rule_prob, token_prob, reference_prob)
    return (out[0], out[1], out[2])


# speedup vs baseline: 34.8233x; 34.8233x over previous
import jax
import jax.numpy as jnp
from jax.experimental import pallas as pl
from jax.experimental.pallas import tpu as pltpu


def _copy3(idx_ref, r_ref, t_ref, p_ref, ro_ref, to_ref, po_ref):
    del idx_ref
    ro_ref[...] = r_ref[0]
    to_ref[...] = t_ref[0]
    po_ref[...] = p_ref[0]


def kernel(rule_prob, token_prob, reference_prob, length):
    L, B, R = rule_prob.shape
    V = token_prob.shape[2]
    M = reference_prob.shape[2]
    idx = (length - 1).astype(jnp.int32)
    tok_t = token_prob.transpose(0, 2, 1)
    ref_t = reference_prob.transpose(0, 2, 1)

    grid_spec = pltpu.PrefetchScalarGridSpec(
        num_scalar_prefetch=1,
        grid=(1,),
        in_specs=[
            pl.BlockSpec((1, B, R), lambda g, idx_ref: (idx_ref[0], 0, 0)),
            pl.BlockSpec((1, V, B), lambda g, idx_ref: (idx_ref[0], 0, 0)),
            pl.BlockSpec((1, M, B), lambda g, idx_ref: (idx_ref[0], 0, 0)),
        ],
        out_specs=[
            pl.BlockSpec((B, R), lambda g, idx_ref: (0, 0)),
            pl.BlockSpec((V, B), lambda g, idx_ref: (0, 0)),
            pl.BlockSpec((M, B), lambda g, idx_ref: (0, 0)),
        ],
    )
    r, t_t, p_t = pl.pallas_call(
        _copy3,
        grid_spec=grid_spec,
        out_shape=[
            jax.ShapeDtypeStruct((B, R), jnp.float32),
            jax.ShapeDtypeStruct((V, B), jnp.float32),
            jax.ShapeDtypeStruct((M, B), jnp.float32),
        ],
    )(idx, rule_prob, tok_t, ref_t)
    return (r, t_t.T, p_t.T)
